# baseline (speedup 1.0000x reference)
import functools

import jax
import jax.numpy as jnp
from jax import lax
from jax.experimental import pallas as pl
from jax.experimental.pallas import tpu as pltpu
from jax.experimental.pallas import tpu_sc as plsc

_NUM_CORES = 2
_NUM_SUBCORES = 16
_NUM_WORKERS = _NUM_CORES * _NUM_SUBCORES

_CHUNK = 64


def _make_gather(batch: int, seq: int, d_model: int, dtype):
    half = _NUM_WORKERS // 2
    n_rows = batch * seq
    rows_per_w = n_rows // half
    n_chunks = rows_per_w // _CHUNK
    n_pairs = n_chunks // 2
    assert n_rows % half == 0 and rows_per_w % _CHUNK == 0 and n_chunks % 2 == 0
    assert rows_per_w <= seq and seq % rows_per_w == 0
    mesh = plsc.VectorSubcoreMesh(
        core_axis_name="c", subcore_axis_name="s",
        num_cores=_NUM_CORES, num_subcores=_NUM_SUBCORES,
    )
    out = jax.ShapeDtypeStruct((batch, seq, d_model), dtype)

    @functools.partial(
        pl.kernel,
        out_type=(out, out),
        mesh=mesh,
        scratch_types=[
            pltpu.VMEM((rows_per_w,), jnp.int32),
            pltpu.VMEM((2, _CHUNK, d_model), dtype),
            pltpu.SemaphoreType.DMA,
            pltpu.SemaphoreType.DMA,
        ],
    )
    def gather_kernel(enc_hbm, dec_hbm, table_hbm, enc_out, dec_out,
                      idx_v, rows_v, gsem, osem):
        wid = lax.axis_index("s") * _NUM_CORES + lax.axis_index("c")

        def run(idx_hbm, out_hbm, slot):
            b = (slot * rows_per_w) // seq
            t0 = (slot * rows_per_w) % seq

            def idx_slice(i):
                return idx_v.at[pl.ds(i * _CHUNK, _CHUNK)]

            def out_slice(i):
                return out_hbm.at[b, pl.ds(t0 + i * _CHUNK, _CHUNK), :]

            pltpu.sync_copy(idx_hbm.at[b, pl.ds(t0, rows_per_w)], idx_v)
            pltpu.async_copy(table_hbm.at[idx_slice(0)], rows_v.at[0], gsem)
            pltpu.async_copy(table_hbm.at[idx_slice(1)], rows_v.at[1], gsem)

            def body(p, _):
                i0 = p * 2
                for j in range(2):
                    pltpu.make_async_copy(
                        table_hbm.at[idx_slice(i0 + j)], rows_v.at[j], gsem
                    ).wait()
                    pltpu.async_copy(rows_v.at[j], out_slice(i0 + j), osem)

                @pl.when(p + 1 < n_pairs)
                def _():
                    for j in range(2):
                        pltpu.make_async_copy(
                            rows_v.at[j], out_slice(i0 + j), osem
                        ).wait()
                        pltpu.async_copy(
                            table_hbm.at[idx_slice(i0 + 2 + j)], rows_v.at[j], gsem
                        )

                return 0

            lax.fori_loop(0, n_pairs, body, 0)

            for j in range(2):
                pltpu.make_async_copy(
                    rows_v.at[j], out_slice(n_chunks - 2 + j), osem
                ).wait()

        @pl.when(wid < half)
        def _():
            run(enc_hbm, enc_out, wid)

        @pl.when(wid >= half)
        def _():
            run(dec_hbm, dec_out, wid - half)

    return gather_kernel


@jax.jit
def kernel(input_ids, decoder_input_ids, table):
    b, s = input_ids.shape
    d = table.shape[1]
    return _make_gather(b, s, d, table.dtype)(
        input_ids.astype(jnp.int32), decoder_input_ids.astype(jnp.int32), table
    )


# speedup vs baseline: 1.0102x; 1.0102x over previous
import functools

import jax
import jax.numpy as jnp
from jax import lax
from jax.experimental import pallas as pl
from jax.experimental.pallas import tpu as pltpu
from jax.experimental.pallas import tpu_sc as plsc

_NUM_CORES = 2
_NUM_SUBCORES = 16
_NUM_WORKERS = _NUM_CORES * _NUM_SUBCORES

_CHUNK = 16
_NBUF = 4


def _make_gather(batch: int, seq: int, d_model: int, dtype):
    half = _NUM_WORKERS // 2
    n_rows = batch * seq
    rows_per_w = n_rows // half
    n_chunks = rows_per_w // _CHUNK
    assert n_rows % half == 0 and rows_per_w % _CHUNK == 0 and n_chunks >= _NBUF
    assert seq % rows_per_w == 0 or rows_per_w % seq == 0
    mesh = plsc.VectorSubcoreMesh(
        core_axis_name="c", subcore_axis_name="s",
        num_cores=_NUM_CORES, num_subcores=_NUM_SUBCORES,
    )
    out = jax.ShapeDtypeStruct((batch, seq, d_model), dtype)

    @functools.partial(
        pl.kernel,
        out_type=(out, out),
        mesh=mesh,
        scratch_types=[
            pltpu.VMEM((rows_per_w,), jnp.int32),
            pltpu.VMEM((_NBUF, _CHUNK, d_model), dtype),
            pltpu.VMEM_SHARED((_NUM_SUBCORES, 2, _CHUNK, d_model), dtype),
            pltpu.SemaphoreType.DMA,
            pltpu.SemaphoreType.DMA,
            pltpu.SemaphoreType.DMA,
        ],
    )
    def gather_kernel(enc_hbm, dec_hbm, table_hbm, enc_out, dec_out,
                      idx_v, rows_v, sp, gsem, hsem, ssem):
        sid = lax.axis_index("s")
        wid = sid * _NUM_CORES + lax.axis_index("c")

        def run(idx_hbm, out_hbm, slot):
            b = (slot * rows_per_w) // seq
            t0 = (slot * rows_per_w) % seq

            def idx_slice(i):
                return idx_v.at[pl.ds(i * _CHUNK, _CHUNK)]

            def out_slice(i):
                return out_hbm.at[b, pl.ds(t0 + i * _CHUNK, _CHUNK), :]

            pltpu.sync_copy(idx_hbm.at[b, pl.ds(t0, rows_per_w)], idx_v)
            for j in range(_NBUF):
                pltpu.async_copy(table_hbm.at[idx_slice(j)], rows_v.at[j], gsem)

            for i in range(n_chunks):
                buf = i % _NBUF
                k = i % 2
                pltpu.make_async_copy(
                    table_hbm.at[idx_slice(i)], rows_v.at[buf], gsem
                ).wait()
                if i >= 2:
                    pltpu.make_async_copy(
                        sp.at[sid, k], out_slice(i - 2), ssem
                    ).wait()
                pltpu.make_async_copy(rows_v.at[buf], sp.at[sid, k], hsem).start()
                pltpu.make_async_copy(rows_v.at[buf], sp.at[sid, k], hsem).wait()
                pltpu.async_copy(sp.at[sid, k], out_slice(i), ssem)
                if i + _NBUF < n_chunks:
                    pltpu.async_copy(
                        table_hbm.at[idx_slice(i + _NBUF)], rows_v.at[buf], gsem
                    )

            for i in (n_chunks - 2, n_chunks - 1):
                pltpu.make_async_copy(
                    sp.at[sid, i % 2], out_slice(i), ssem
                ).wait()

        @pl.when(wid < half)
        def _():
            run(enc_hbm, enc_out, wid)

        @pl.when(wid >= half)
        def _():
            run(dec_hbm, dec_out, wid - half)

    return gather_kernel


@jax.jit
def kernel(input_ids, decoder_input_ids, table):
    b, s = input_ids.shape
    d = table.shape[1]
    return _make_gather(b, s, d, table.dtype)(
        input_ids.astype(jnp.int32), decoder_input_ids.astype(jnp.int32), table
    )
